# baseline (speedup 1.0000x reference)
import functools

import jax
import jax.numpy as jnp
from jax import lax
from jax.experimental import pallas as pl
from jax.experimental.pallas import tpu as pltpu
from jax.experimental.pallas import tpu_sc as plsc

BATCH = 16384
EMBED_DIM = 64
FEAT_DIM = 64
DENSE_DIM = 32
PAIR_DIM = 2 * EMBED_DIM

NUM_CORES = 2
NUM_SUBCORES = 16
NUM_WORKERS = NUM_CORES * NUM_SUBCORES
B_PER_W = BATCH // NUM_WORKERS
GATHER_CHUNK = 128
N_CHUNKS = B_PER_W // GATHER_CHUNK


def _sc_gather_pair(user_pairs, uid2, item_pairs, vid2):
    mesh = plsc.VectorSubcoreMesh(core_axis_name="c", subcore_axis_name="s")
    out_t = (
        jax.ShapeDtypeStruct((BATCH, PAIR_DIM), jnp.float32),
        jax.ShapeDtypeStruct((BATCH, PAIR_DIM), jnp.float32),
    )

    @functools.partial(
        pl.kernel,
        out_type=out_t,
        mesh=mesh,
        scratch_types=[
            pltpu.VMEM((B_PER_W,), jnp.int32),
            pltpu.VMEM((B_PER_W,), jnp.int32),
            pltpu.VMEM((2, GATHER_CHUNK, PAIR_DIM), jnp.float32),
            pltpu.VMEM((2, GATHER_CHUNK, PAIR_DIM), jnp.float32),
            pltpu.SemaphoreType.DMA,
            pltpu.SemaphoreType.DMA,
            pltpu.SemaphoreType.DMA,
            pltpu.SemaphoreType.DMA,
        ],
    )
    def k(ut_hbm, uid_hbm, it_hbm, vid_hbm, uout_hbm, iout_hbm,
          idx_u, idx_i, rows_u, rows_i, su0, su1, si0, si1):
        wid = lax.axis_index("s") * NUM_CORES + lax.axis_index("c")
        base = wid * B_PER_W
        sems_u = (su0, su1)
        sems_i = (si0, si1)
        pltpu.sync_copy(uid_hbm.at[pl.ds(base, B_PER_W)], idx_u)
        pltpu.sync_copy(vid_hbm.at[pl.ds(base, B_PER_W)], idx_i)

        def fire(c):
            b = c % 2
            sl = pl.ds(c * GATHER_CHUNK, GATHER_CHUNK)
            pltpu.async_copy(ut_hbm.at[idx_u.at[sl]], rows_u.at[b], sems_u[b])
            pltpu.async_copy(it_hbm.at[idx_i.at[sl]], rows_i.at[b], sems_i[b])

        fire(0)
        fire(1)
        for c in range(N_CHUNKS):
            b = c % 2
            sl = pl.ds(c * GATHER_CHUNK, GATHER_CHUNK)
            osl = pl.ds(base + c * GATHER_CHUNK, GATHER_CHUNK)
            pltpu.make_async_copy(ut_hbm.at[idx_u.at[sl]], rows_u.at[b],
                                  sems_u[b]).wait()
            pltpu.sync_copy(rows_u.at[b], uout_hbm.at[osl])
            pltpu.make_async_copy(it_hbm.at[idx_i.at[sl]], rows_i.at[b],
                                  sems_i[b]).wait()
            pltpu.sync_copy(rows_i.at[b], iout_hbm.at[osl])
            if c + 2 < N_CHUNKS:
                fire(c + 2)

    return k(user_pairs, uid2, item_pairs, vid2)


BLK = 2048


def _tc_body(up_ref, ip_ref, pu_ref, pi_ref, uf_ref, vf_ref,
             wu_ref, bu_ref, wi_ref, bi_ref, out_ref):
    up = up_ref[...]
    ip = ip_ref[...]
    pu = pu_ref[...]
    pi = pi_ref[...]
    u_emb = up[:, :EMBED_DIM] + pu * (up[:, EMBED_DIM:] - up[:, :EMBED_DIM])
    i_emb = ip[:, :EMBED_DIM] + pi * (ip[:, EMBED_DIM:] - ip[:, :EMBED_DIM])
    u_feat = jnp.maximum(
        jnp.dot(uf_ref[...], wu_ref[...],
                preferred_element_type=jnp.float32) + bu_ref[...], 0.0)
    i_feat = jnp.maximum(
        jnp.dot(vf_ref[...], wi_ref[...],
                preferred_element_type=jnp.float32) + bi_ref[...], 0.0)
    dot = (jnp.sum(u_emb * i_emb, axis=1) + jnp.sum(u_feat * i_feat, axis=1))
    out_ref[...] = dot[None, :]


def _tc_combine(u_pair, i_pair, pu, pi, user_features, video_features,
                Wu, bu, Wi, bi):
    grid = (BATCH // BLK,)
    bspec_pair = pl.BlockSpec((BLK, PAIR_DIM), lambda i: (i, 0))
    bspec_par = pl.BlockSpec((BLK, 1), lambda i: (i, 0))
    bspec_b = pl.BlockSpec((BLK, FEAT_DIM), lambda i: (i, 0))
    bspec_w = pl.BlockSpec((FEAT_DIM, DENSE_DIM), lambda i: (0, 0))
    bspec_bias = pl.BlockSpec((1, DENSE_DIM), lambda i: (0, 0))
    out = pl.pallas_call(
        _tc_body,
        grid=grid,
        in_specs=[bspec_pair, bspec_pair, bspec_par, bspec_par,
                  bspec_b, bspec_b, bspec_w, bspec_bias, bspec_w, bspec_bias],
        out_specs=pl.BlockSpec((1, BLK), lambda i: (0, i)),
        out_shape=jax.ShapeDtypeStruct((1, BATCH), jnp.float32),
    )(u_pair, i_pair, pu, pi, user_features, video_features,
      Wu, bu.reshape(1, DENSE_DIM), Wi, bi.reshape(1, DENSE_DIM))
    return out.reshape(BATCH)


@jax.jit
def kernel(user_id, user_features, video_id, video_features, user_table,
           item_table, Wu, bu, Wi, bi):
    uid = user_id.astype(jnp.int32)
    vid = video_id.astype(jnp.int32)
    user_pairs = user_table.reshape(-1, PAIR_DIM)
    item_pairs = item_table.reshape(-1, PAIR_DIM)
    pu = (uid & 1).astype(jnp.float32).reshape(BATCH, 1)
    pi = (vid & 1).astype(jnp.float32).reshape(BATCH, 1)
    u_pair, i_pair = _sc_gather_pair(user_pairs, uid >> 1, item_pairs,
                                     vid >> 1)
    return _tc_combine(u_pair, i_pair, pu, pi, user_features, video_features,
                       Wu, bu, Wi, bi)


# speedup vs baseline: 1.9827x; 1.9827x over previous
import dataclasses
import functools

import jax
import jax.numpy as jnp
from jax import lax
from jax.experimental import pallas as pl
from jax.experimental.pallas import tpu as pltpu
from jax.experimental.pallas import tpu_sc as plsc

BATCH = 16384
EMBED_DIM = 64
FEAT_DIM = 64
DENSE_DIM = 32
NUM_ROWS = 1000000

NUM_CORES = 2
NUM_SUBCORES = 16
NUM_WORKERS = NUM_CORES * NUM_SUBCORES

WIN = 512
RANGE_PER_W = 31232
N_WIN = 62
TAIL_START = 999936
LOC_CAP = 2048
WCAP = 48
OUT_ROWS = BATCH + 16


def _sc_stream_gather(user_t, uid, item_t, vid, tail_u, tail_i):
    mesh = plsc.VectorSubcoreMesh(core_axis_name="c", subcore_axis_name="s")
    out_t = (
        jax.ShapeDtypeStruct((OUT_ROWS, 2 * EMBED_DIM), jnp.float32),
        jax.ShapeDtypeStruct((OUT_ROWS, 2 * EMBED_DIM), jnp.float32),
    )

    cp = pltpu.CompilerParams()
    if "needs_layout_passes" in pltpu.CompilerParams.__dataclass_fields__:
        cp = dataclasses.replace(cp, needs_layout_passes=False)

    @functools.partial(
        pl.kernel,
        out_type=out_t,
        mesh=mesh,
        compiler_params=cp,
        scratch_types=[
            pltpu.VMEM((BATCH,), jnp.int32),
            pltpu.VMEM((LOC_CAP + 16,), jnp.int32),
            pltpu.VMEM((LOC_CAP + 16,), jnp.int32),
            pltpu.VMEM((EMBED_DIM, WIN), jnp.float32),
            pltpu.VMEM((WCAP + 16,), jnp.int32),
            pltpu.VMEM((WCAP + 16,), jnp.int32),
            pltpu.VMEM((16, 2 * EMBED_DIM), jnp.float32),
            pltpu.VMEM((16,), jnp.int32),
            pltpu.VMEM((EMBED_DIM, NUM_ROWS - TAIL_START), jnp.float32),
        ],
    )
    def k(ut_hbm, uid_hbm, it_hbm, vid_hbm, tu_hbm, ti_hbm,
          uout_hbm, iout_hbm,
          ids_v, loc_u, loc_p, win_v, wu_v, wp_v, stage_v, pos_v, tail_v):
        wid = lax.axis_index("s") * NUM_CORES + lax.axis_index("c")
        lo = wid * RANGE_PER_W
        is_last = wid == NUM_WORKERS - 1
        hi_list = jnp.where(is_last, NUM_ROWS, lo + RANGE_PER_W)
        win_hi = jnp.where(is_last, TAIL_START, lo + RANGE_PER_W)
        iota16 = lax.iota(jnp.int32, 16)

        def compact_ids(lo_s, hi_s):
            lo_v = jnp.full((16,), lo_s, jnp.int32)
            hi_v = jnp.full((16,), hi_s, jnp.int32)

            def body(j, cnt):
                u = ids_v[pl.ds(j * 16, 16)]
                pos = jnp.full((16,), j * 16, jnp.int32) + iota16
                m = (u >= lo_v) & (u < hi_v)
                inc = plsc.cumsum(m.astype(jnp.int32))
                idx = jnp.full((16,), cnt, jnp.int32) + inc - 1
                plsc.store_scatter(loc_u, [idx], u, mask=m)
                plsc.store_scatter(loc_p, [idx], pos, mask=m)
                return jnp.minimum(cnt + jnp.max(inc), LOC_CAP)

            return lax.fori_loop(0, BATCH // 16, body, jnp.int32(0))

        def window(buf_v, o_hbm, cnt, s, width):
            s_v = jnp.full((16,), s, jnp.int32)
            w_v = jnp.full((16,), width, jnp.int32)
            cnt_v = jnp.full((16,), cnt, jnp.int32)

            def scan(kk, wcnt):
                lu = loc_u[pl.ds(kk * 16, 16)]
                lp = loc_p[pl.ds(kk * 16, 16)]
                valid = (jnp.full((16,), kk * 16, jnp.int32) + iota16) < cnt_v
                m = valid & (lu >= s_v) & (lu < s_v + w_v)
                inc = plsc.cumsum(m.astype(jnp.int32))
                idx = jnp.full((16,), wcnt, jnp.int32) + inc - 1
                plsc.store_scatter(wu_v, [idx], lu, mask=m)
                plsc.store_scatter(wp_v, [idx], lp, mask=m)
                return jnp.minimum(wcnt + jnp.max(inc), WCAP)

            wcnt = lax.fori_loop(0, (cnt + 15) // 16, scan, jnp.int32(0))
            wcnt_v = jnp.full((16,), wcnt, jnp.int32)
            for q in range(WCAP // 16):
                @pl.when(q * 16 < wcnt)
                def _():
                    cu = wu_v[pl.ds(q * 16, 16)]
                    cp = wp_v[pl.ds(q * 16, 16)]
                    vm = (jnp.full((16,), q * 16, jnp.int32) + iota16) < wcnt_v
                    lane = jnp.where(vm, cu - s_v, 0)
                    pos_v[...] = jnp.where(vm, cp,
                                           jnp.full((16,), BATCH, jnp.int32)
                                           + iota16)

                    @pl.loop(0, EMBED_DIM)
                    def _(d):
                        d_v = jnp.full((16,), d, jnp.int32)
                        vals = plsc.load_gather(buf_v, [d_v, lane])
                        plsc.store_scatter(stage_v, [iota16, d_v], vals)

                    pltpu.sync_copy(stage_v, o_hbm.at[pos_v])

        for t_hbm, id_hbm, t_tail, o_hbm in (
                (ut_hbm, uid_hbm, tu_hbm, uout_hbm),
                (it_hbm, vid_hbm, ti_hbm, iout_hbm)):
            pltpu.sync_copy(id_hbm, ids_v)
            cnt = compact_ids(lo, hi_list)

            @pl.loop(0, N_WIN)
            def _(i):
                s = lo + i * WIN

                @pl.when(s < win_hi)
                def _():
                    s_al = pl.multiple_of(s, 128)
                    pltpu.sync_copy(t_hbm.at[:, pl.ds(s_al, WIN)], win_v)
                    window(win_v, o_hbm, cnt, s, WIN)

            @pl.when(is_last)
            def _():
                pltpu.sync_copy(t_tail, tail_v)
                window(tail_v, o_hbm, cnt, jnp.int32(TAIL_START),
                       NUM_ROWS - TAIL_START)

    return k(user_t, uid, item_t, vid, tail_u, tail_i)


BLK = 2048


def _tc_body(ue_ref, ie_ref, uf_ref, vf_ref, wu_ref, bu_ref, wi_ref, bi_ref,
             out_ref):
    u_emb = ue_ref[:, :EMBED_DIM]
    i_emb = ie_ref[:, :EMBED_DIM]
    u_feat = jnp.maximum(
        jnp.dot(uf_ref[...], wu_ref[...],
                preferred_element_type=jnp.float32) + bu_ref[...], 0.0)
    i_feat = jnp.maximum(
        jnp.dot(vf_ref[...], wi_ref[...],
                preferred_element_type=jnp.float32) + bi_ref[...], 0.0)
    dot = (jnp.sum(u_emb * i_emb, axis=1) + jnp.sum(u_feat * i_feat, axis=1))
    out_ref[...] = dot[None, :]


def _tc_combine(u_rows, i_rows, user_features, video_features, Wu, bu, Wi, bi):
    grid = (BATCH // BLK,)
    bspec_rows = pl.BlockSpec((BLK, 2 * EMBED_DIM), lambda i: (i, 0))
    bspec_b = pl.BlockSpec((BLK, FEAT_DIM), lambda i: (i, 0))
    bspec_w = pl.BlockSpec((FEAT_DIM, DENSE_DIM), lambda i: (0, 0))
    bspec_bias = pl.BlockSpec((1, DENSE_DIM), lambda i: (0, 0))
    out = pl.pallas_call(
        _tc_body,
        grid=grid,
        in_specs=[bspec_rows, bspec_rows, bspec_b, bspec_b,
                  bspec_w, bspec_bias, bspec_w, bspec_bias],
        out_specs=pl.BlockSpec((1, BLK), lambda i: (0, i)),
        out_shape=jax.ShapeDtypeStruct((1, BATCH), jnp.float32),
    )(u_rows, i_rows, user_features, video_features,
      Wu, bu.reshape(1, DENSE_DIM), Wi, bi.reshape(1, DENSE_DIM))
    return out.reshape(BATCH)


@jax.jit
def kernel(user_id, user_features, video_id, video_features, user_table,
           item_table, Wu, bu, Wi, bi):
    uid = user_id.astype(jnp.int32)
    vid = video_id.astype(jnp.int32)
    u_rows, i_rows = _sc_stream_gather(
        user_table.T, uid, item_table.T, vid,
        user_table[TAIL_START:].T, item_table[TAIL_START:].T)
    return _tc_combine(u_rows, i_rows, user_features, video_features,
                       Wu, bu, Wi, bi)


# speedup vs baseline: 2.0229x; 1.0203x over previous
import dataclasses
import functools

import jax
import jax.numpy as jnp
from jax import lax
from jax.experimental import pallas as pl
from jax.experimental.pallas import tpu as pltpu
from jax.experimental.pallas import tpu_sc as plsc

BATCH = 16384
EMBED_DIM = 64
FEAT_DIM = 64
DENSE_DIM = 32
NUM_ROWS = 1000000

NUM_CORES = 2
NUM_SUBCORES = 16
NUM_WORKERS = NUM_CORES * NUM_SUBCORES

WIN = 512
RANGE_PER_W = 31232
N_WIN = 62
TAIL_START = 999936
LOC_CAP = 2048
WCAP = 48
OUT_ROWS = BATCH + 16


def _sc_stream_gather(user_t, uid, item_t, vid, tail_u, tail_i):
    mesh = plsc.VectorSubcoreMesh(core_axis_name="c", subcore_axis_name="s")
    out_t = (
        jax.ShapeDtypeStruct((OUT_ROWS, 2 * EMBED_DIM), jnp.float32),
        jax.ShapeDtypeStruct((OUT_ROWS, 2 * EMBED_DIM), jnp.float32),
    )

    cp = pltpu.CompilerParams()
    if "needs_layout_passes" in pltpu.CompilerParams.__dataclass_fields__:
        cp = dataclasses.replace(cp, needs_layout_passes=False)

    @functools.partial(
        pl.kernel,
        out_type=out_t,
        mesh=mesh,
        compiler_params=cp,
        scratch_types=[
            pltpu.VMEM((BATCH,), jnp.int32),
            pltpu.VMEM((LOC_CAP + 16,), jnp.int32),
            pltpu.VMEM((LOC_CAP + 16,), jnp.int32),
            pltpu.VMEM((EMBED_DIM, WIN), jnp.float32),
            pltpu.VMEM((EMBED_DIM, WIN), jnp.float32),
            pltpu.SemaphoreType.DMA,
            pltpu.SemaphoreType.DMA,
            pltpu.VMEM((WCAP + 16,), jnp.int32),
            pltpu.VMEM((WCAP + 16,), jnp.int32),
            pltpu.VMEM((16, 2 * EMBED_DIM), jnp.float32),
            pltpu.VMEM((16,), jnp.int32),
            pltpu.VMEM((EMBED_DIM, NUM_ROWS - TAIL_START), jnp.float32),
        ],
    )
    def k(ut_hbm, uid_hbm, it_hbm, vid_hbm, tu_hbm, ti_hbm,
          uout_hbm, iout_hbm,
          ids_v, loc_u, loc_p, win_a, win_b, sem_a, sem_b,
          wu_v, wp_v, stage_v, pos_v, tail_v):
        wid = lax.axis_index("s") * NUM_CORES + lax.axis_index("c")
        lo = wid * RANGE_PER_W
        is_last = wid == NUM_WORKERS - 1
        hi_list = jnp.where(is_last, NUM_ROWS, lo + RANGE_PER_W)
        win_hi = jnp.where(is_last, TAIL_START, lo + RANGE_PER_W)
        iota16 = lax.iota(jnp.int32, 16)

        def compact_ids(lo_s, hi_s):
            lo_v = jnp.full((16,), lo_s, jnp.int32)
            hi_v = jnp.full((16,), hi_s, jnp.int32)

            def body(j, cnt):
                u = ids_v[pl.ds(j * 16, 16)]
                pos = jnp.full((16,), j * 16, jnp.int32) + iota16
                m = (u >= lo_v) & (u < hi_v)
                inc = plsc.cumsum(m.astype(jnp.int32))
                idx = jnp.full((16,), cnt, jnp.int32) + inc - 1
                plsc.store_scatter(loc_u, [idx], u, mask=m)
                plsc.store_scatter(loc_p, [idx], pos, mask=m)
                return jnp.minimum(cnt + jnp.max(inc), LOC_CAP)

            return lax.fori_loop(0, BATCH // 16, body, jnp.int32(0))

        def window(buf_v, o_hbm, cnt, s, width):
            s_v = jnp.full((16,), s, jnp.int32)
            w_v = jnp.full((16,), width, jnp.int32)
            cnt_v = jnp.full((16,), cnt, jnp.int32)

            def scan(kk, wcnt):
                lu = loc_u[pl.ds(kk * 16, 16)]
                lp = loc_p[pl.ds(kk * 16, 16)]
                valid = (jnp.full((16,), kk * 16, jnp.int32) + iota16) < cnt_v
                m = valid & (lu >= s_v) & (lu < s_v + w_v)
                inc = plsc.cumsum(m.astype(jnp.int32))
                idx = jnp.full((16,), wcnt, jnp.int32) + inc - 1
                plsc.store_scatter(wu_v, [idx], lu, mask=m)
                plsc.store_scatter(wp_v, [idx], lp, mask=m)
                return jnp.minimum(wcnt + jnp.max(inc), WCAP)

            wcnt = lax.fori_loop(0, (cnt + 15) // 16, scan, jnp.int32(0))
            wcnt_v = jnp.full((16,), wcnt, jnp.int32)
            for q in range(WCAP // 16):
                @pl.when(q * 16 < wcnt)
                def _():
                    cu = wu_v[pl.ds(q * 16, 16)]
                    cp = wp_v[pl.ds(q * 16, 16)]
                    vm = (jnp.full((16,), q * 16, jnp.int32) + iota16) < wcnt_v
                    lane = jnp.where(vm, cu - s_v, 0)
                    pos_v[...] = jnp.where(vm, cp,
                                           jnp.full((16,), BATCH, jnp.int32)
                                           + iota16)

                    @pl.loop(0, EMBED_DIM)
                    def _(d):
                        d_v = jnp.full((16,), d, jnp.int32)
                        vals = plsc.load_gather(buf_v, [d_v, lane])
                        plsc.store_scatter(stage_v, [iota16, d_v], vals)

                    pltpu.sync_copy(stage_v, o_hbm.at[pos_v])

        for t_hbm, id_hbm, t_tail, o_hbm in (
                (ut_hbm, uid_hbm, tu_hbm, uout_hbm),
                (it_hbm, vid_hbm, ti_hbm, iout_hbm)):
            pltpu.sync_copy(id_hbm, ids_v)
            cnt = compact_ids(lo, hi_list)

            def wslice(s):
                return t_hbm.at[:, pl.ds(pl.multiple_of(s, 128), WIN)]

            def fire(s, buf, sem):
                pltpu.async_copy(wslice(s), buf, sem)

            def drain(s, buf, sem):
                pltpu.make_async_copy(wslice(s), buf, sem).wait()

            fire(lo, win_a, sem_a)

            @pl.loop(0, N_WIN // 2)
            def _(j):
                s0 = lo + (2 * j) * WIN
                s1 = s0 + WIN
                s2 = s1 + WIN

                @pl.when(s1 < win_hi)
                def _():
                    fire(s1, win_b, sem_b)

                @pl.when(s0 < win_hi)
                def _():
                    drain(s0, win_a, sem_a)
                    window(win_a, o_hbm, cnt, s0, WIN)

                @pl.when(s2 < win_hi)
                def _():
                    fire(s2, win_a, sem_a)

                @pl.when(s1 < win_hi)
                def _():
                    drain(s1, win_b, sem_b)
                    window(win_b, o_hbm, cnt, s1, WIN)

            @pl.when(is_last)
            def _():
                pltpu.sync_copy(t_tail, tail_v)
                window(tail_v, o_hbm, cnt, jnp.int32(TAIL_START),
                       NUM_ROWS - TAIL_START)

    return k(user_t, uid, item_t, vid, tail_u, tail_i)


BLK = 2048


def _tc_body(ue_ref, ie_ref, uf_ref, vf_ref, wu_ref, bu_ref, wi_ref, bi_ref,
             out_ref):
    u_emb = ue_ref[:, :EMBED_DIM]
    i_emb = ie_ref[:, :EMBED_DIM]
    u_feat = jnp.maximum(
        jnp.dot(uf_ref[...], wu_ref[...],
                preferred_element_type=jnp.float32) + bu_ref[...], 0.0)
    i_feat = jnp.maximum(
        jnp.dot(vf_ref[...], wi_ref[...],
                preferred_element_type=jnp.float32) + bi_ref[...], 0.0)
    dot = (jnp.sum(u_emb * i_emb, axis=1) + jnp.sum(u_feat * i_feat, axis=1))
    out_ref[...] = dot[None, :]


def _tc_combine(u_rows, i_rows, user_features, video_features, Wu, bu, Wi, bi):
    grid = (BATCH // BLK,)
    bspec_rows = pl.BlockSpec((BLK, 2 * EMBED_DIM), lambda i: (i, 0))
    bspec_b = pl.BlockSpec((BLK, FEAT_DIM), lambda i: (i, 0))
    bspec_w = pl.BlockSpec((FEAT_DIM, DENSE_DIM), lambda i: (0, 0))
    bspec_bias = pl.BlockSpec((1, DENSE_DIM), lambda i: (0, 0))
    out = pl.pallas_call(
        _tc_body,
        grid=grid,
        in_specs=[bspec_rows, bspec_rows, bspec_b, bspec_b,
                  bspec_w, bspec_bias, bspec_w, bspec_bias],
        out_specs=pl.BlockSpec((1, BLK), lambda i: (0, i)),
        out_shape=jax.ShapeDtypeStruct((1, BATCH), jnp.float32),
    )(u_rows, i_rows, user_features, video_features,
      Wu, bu.reshape(1, DENSE_DIM), Wi, bi.reshape(1, DENSE_DIM))
    return out.reshape(BATCH)


@jax.jit
def kernel(user_id, user_features, video_id, video_features, user_table,
           item_table, Wu, bu, Wi, bi):
    uid = user_id.astype(jnp.int32)
    vid = video_id.astype(jnp.int32)
    u_rows, i_rows = _sc_stream_gather(
        user_table.T, uid, item_table.T, vid,
        user_table[TAIL_START:].T, item_table[TAIL_START:].T)
    return _tc_combine(u_rows, i_rows, user_features, video_features,
                       Wu, bu, Wi, bi)
